# baseline (speedup 1.0000x reference)
import functools
import jax
import jax.numpy as jnp
from jax import lax
from jax.experimental import pallas as pl
from jax.experimental.pallas import tpu as pltpu
from jax.experimental.pallas import tpu_sc as plsc

HS = 2048
NB = 32
RANK = 8
GRID_N = 8
SIGMA = 1.0
ROW_TILE = 1024
QPAD = 8

_TRANS_B = (((1,), (1,)), ((), ()))


def _phase_a(x_ref, prior_ref, wproj_ref, sb_ref, caug_ref, down_ref,
             scal_ref, scores_ref, z_ref):
    x = x_ref[...]
    rps = scal_ref[0, 0]
    xq = lax.dot_general(x, wproj_ref[...], _TRANS_B,
                         preferred_element_type=jnp.float32)
    mu = xq[:, 3:4] * (1.0 / HS)
    var = jnp.mean(x * x, axis=1, keepdims=True) - mu * mu
    rs = lax.rsqrt(var + 1e-5)
    qraw = rs * (xq - mu * sb_ref[1:2, :]) + sb_ref[0:1, :]
    col = lax.broadcasted_iota(jnp.int32, qraw.shape, 1)
    q = jnp.where(col < 3, jax.nn.sigmoid(qraw) * float(GRID_N - 1), 0.0)
    qn = jnp.sum(q * q, axis=1, keepdims=True)
    q_aug = q + (col == 3).astype(jnp.float32)
    d2 = qn + lax.dot_general(q_aug, caug_ref[...], _TRANS_B,
                              preferred_element_type=jnp.float32)
    spatial = jnp.exp(d2 * (-1.0 / (2.0 * SIGMA * SIGMA)))
    prior = jnp.maximum(prior_ref[...], 0.0)
    prior = prior / jnp.maximum(jnp.sum(prior, axis=1, keepdims=True), 1e-6)
    prior_bias = jnp.clip(jnp.log(prior + 1e-6), -6.0, 0.0)
    scores_ref[...] = jnp.transpose(spatial + rps * prior_bias)
    z_ref[...] = jnp.dot(x, down_ref[...], preferred_element_type=jnp.float32)


def _phase_c(x_ref, z_ref, w_ref, up_ref, scal_ref, out_ref):
    x = x_ref[...]
    res = scal_ref[0, 1]
    wfull = jnp.transpose(w_ref[...])
    erow = lax.broadcasted_iota(jnp.int32, (NB, RANK * NB), 0)
    ecol = lax.broadcasted_iota(jnp.int32, (NB, RANK * NB), 1)
    expand = (erow == ecol // RANK).astype(jnp.float32)
    wexp = jnp.dot(wfull, expand, preferred_element_type=jnp.float32)
    delta = jnp.dot(z_ref[...] * wexp, up_ref[...],
                    preferred_element_type=jnp.float32)
    out_ref[...] = x + res * delta


def _make_sc_router(rows):
    info = plsc.get_sparse_core_info()
    nc, ns, lanes = info.num_cores, info.num_subcores, info.num_lanes
    nw = nc * ns
    chunk = rows // nw
    groups = chunk // lanes
    mesh = plsc.VectorSubcoreMesh(core_axis_name="c", subcore_axis_name="s")

    @functools.partial(
        pl.kernel, mesh=mesh,
        out_type=jax.ShapeDtypeStruct((NB, rows), jnp.float32),
        scratch_types=[
            pltpu.VMEM((NB, chunk), jnp.float32),
            pltpu.VMEM((NB, chunk), jnp.float32),
        ],
    )
    def router(scores_hbm, out_hbm, sc_v, w_v):
        wid = lax.axis_index("s") * nc + lax.axis_index("c")
        base = wid * chunk
        pltpu.sync_copy(scores_hbm.at[:, pl.ds(base, chunk)], sc_v)

        def per_group(g, carry):
            off = g * lanes
            neg = jnp.full((lanes,), -1e30, jnp.float32)
            zero_i = jnp.zeros((lanes,), jnp.int32)
            m1, i1, m2, i2 = neg, zero_i, neg, zero_i
            for b in range(NB):
                sb = sc_v[b, pl.ds(off, lanes)]
                gt1 = sb > m1
                gt2 = sb > m2
                nm2 = jnp.where(gt1, m1, jnp.where(gt2, sb, m2))
                ni2 = jnp.where(gt1, i1, jnp.where(gt2, b, i2))
                m1 = jnp.where(gt1, sb, m1)
                i1 = jnp.where(gt1, b, i1)
                m2, i2 = nm2, ni2
            e2 = jnp.exp(m2 - m1)
            w1 = 1.0 / (1.0 + e2)
            w2 = e2 * w1
            for b in range(NB):
                wb = (jnp.where(i1 == b, w1, 0.0)
                      + jnp.where(i2 == b, w2, 0.0))
                w_v[b, pl.ds(off, lanes)] = wb
            return carry

        lax.fori_loop(0, groups, per_group, 0)
        pltpu.sync_copy(w_v, out_hbm.at[:, pl.ds(base, chunk)])

    return router


def kernel(hidden_states, route_prior, W_proj, b_proj, block_centers, down_w,
           up_w, route_prior_scale, residual_scale):
    b, s, h = hidden_states.shape
    rows = b * s
    flat = hidden_states.reshape(rows, h)

    wp = jnp.pad(W_proj, ((0, QPAD - 3), (0, 0)))
    wp = wp.at[3, :].set(1.0)
    bias_pad = jnp.pad(b_proj, (0, QPAD - 3)).reshape(1, QPAD)
    colsum = jnp.pad(jnp.sum(W_proj, axis=1), (0, QPAD - 3)).reshape(1, QPAD)
    sb = jnp.concatenate([bias_pad, colsum], axis=0)
    caug = jnp.concatenate(
        [-2.0 * block_centers,
         jnp.sum(block_centers * block_centers, axis=1, keepdims=True),
         jnp.zeros((NB, QPAD - 4), jnp.float32)], axis=1)
    down_all = down_w.transpose(1, 0, 2).reshape(h, RANK * NB)
    up_all = up_w.reshape(RANK * NB, h)
    scal = jnp.stack([route_prior_scale, residual_scale]).reshape(1, 2).astype(jnp.float32)

    grid = rows // ROW_TILE

    scores, z = pl.pallas_call(
        _phase_a,
        grid=(grid,),
        in_specs=[
            pl.BlockSpec((ROW_TILE, h), lambda i: (i, 0)),
            pl.BlockSpec((ROW_TILE, NB), lambda i: (i, 0)),
            pl.BlockSpec((QPAD, h), lambda i: (0, 0)),
            pl.BlockSpec((2, QPAD), lambda i: (0, 0)),
            pl.BlockSpec((NB, QPAD), lambda i: (0, 0)),
            pl.BlockSpec((h, RANK * NB), lambda i: (0, 0)),
            pl.BlockSpec((1, 2), lambda i: (0, 0)),
        ],
        out_specs=[
            pl.BlockSpec((NB, ROW_TILE), lambda i: (0, i)),
            pl.BlockSpec((ROW_TILE, RANK * NB), lambda i: (i, 0)),
        ],
        out_shape=[
            jax.ShapeDtypeStruct((NB, rows), jnp.float32),
            jax.ShapeDtypeStruct((rows, RANK * NB), jnp.float32),
        ],
    )(flat, route_prior, wp, sb, caug, down_all, scal)

    wfull_t = _make_sc_router(rows)(scores)

    out = pl.pallas_call(
        _phase_c,
        grid=(grid,),
        in_specs=[
            pl.BlockSpec((ROW_TILE, h), lambda i: (i, 0)),
            pl.BlockSpec((ROW_TILE, RANK * NB), lambda i: (i, 0)),
            pl.BlockSpec((NB, ROW_TILE), lambda i: (0, i)),
            pl.BlockSpec((RANK * NB, h), lambda i: (0, 0)),
            pl.BlockSpec((1, 2), lambda i: (0, 0)),
        ],
        out_specs=pl.BlockSpec((ROW_TILE, h), lambda i: (i, 0)),
        out_shape=jax.ShapeDtypeStruct((rows, h), jnp.float32),
    )(flat, z, wfull_t, up_all, scal)

    return out.reshape(b, s, h)


# speedup vs baseline: 1.7633x; 1.7633x over previous
import jax
import jax.numpy as jnp
from jax.experimental import pallas as pl

HS = 2048
NB = 32
RANK = 8
GRID_N = 8
SIGMA = 1.0
ROW_TILE = 1024
QPAD = 8

_TRANS_B = (((1,), (1,)), ((), ()))


def _fused_kernel(x_ref, prior_ref, wproj_ref, sb_ref, caug_ref,
                  down_ref, up_ref, out_ref):
    x = x_ref[...]
    rps = sb_ref[2, 0]
    res = sb_ref[2, 1]

    xq = jax.lax.dot_general(x, wproj_ref[...], _TRANS_B,
                             preferred_element_type=jnp.float32)
    mu = xq[:, 3:4] * (1.0 / HS)
    var = jnp.mean(x * x, axis=1, keepdims=True) - mu * mu
    rs = jax.lax.rsqrt(var + 1e-5)
    qraw = rs * (xq - mu * sb_ref[1:2, :]) + sb_ref[0:1, :]
    col = jax.lax.broadcasted_iota(jnp.int32, qraw.shape, 1)
    q = jnp.where(col < 3, jax.nn.sigmoid(qraw) * float(GRID_N - 1), 0.0)
    qn = jnp.sum(q * q, axis=1, keepdims=True)
    q_aug = q + (col == 3).astype(jnp.float32)
    d2 = qn + jax.lax.dot_general(q_aug, caug_ref[...], _TRANS_B,
                                  preferred_element_type=jnp.float32)
    spatial = jnp.exp(d2 * (-1.0 / (2.0 * SIGMA * SIGMA)))

    prior = jnp.maximum(prior_ref[...], 0.0)
    prior = prior / jnp.maximum(jnp.sum(prior, axis=1, keepdims=True), 1e-6)
    prior_bias = jnp.clip(jnp.log(prior + 1e-6), -6.0, 0.0)
    fused = spatial + rps * prior_bias

    iota = jax.lax.broadcasted_iota(jnp.int32, fused.shape, 1)
    m1 = jnp.max(fused, axis=1, keepdims=True)
    i1 = jnp.min(jnp.where(fused == m1, iota, NB), axis=1, keepdims=True)
    oh1 = iota == i1
    masked = jnp.where(oh1, -jnp.inf, fused)
    m2 = jnp.max(masked, axis=1, keepdims=True)
    i2 = jnp.min(jnp.where(masked == m2, iota, NB), axis=1, keepdims=True)
    oh2 = iota == i2
    e2 = jnp.exp(m2 - m1)
    w1 = 1.0 / (1.0 + e2)
    w2 = e2 * w1
    wfull = jnp.where(oh1, w1, 0.0) + jnp.where(oh2, w2, 0.0)

    z = jnp.dot(x, down_ref[...], preferred_element_type=jnp.float32)
    erow = jax.lax.broadcasted_iota(jnp.int32, (NB, RANK * NB), 0)
    ecol = jax.lax.broadcasted_iota(jnp.int32, (NB, RANK * NB), 1)
    expand = (erow == ecol // RANK).astype(jnp.float32)
    wexp = jnp.dot(wfull, expand, preferred_element_type=jnp.float32)
    delta = jnp.dot(z * wexp, up_ref[...], preferred_element_type=jnp.float32)
    out_ref[...] = x + res * delta


def kernel(hidden_states, route_prior, W_proj, b_proj, block_centers, down_w,
           up_w, route_prior_scale, residual_scale):
    b, s, h = hidden_states.shape
    rows = b * s
    flat = hidden_states.reshape(rows, h)

    wp = jnp.concatenate(
        [W_proj, jnp.ones((1, h), jnp.float32),
         jnp.zeros((QPAD - 4, h), jnp.float32)], axis=0)
    zero5 = jnp.zeros((5,), jnp.float32)
    sb = jnp.stack([
        jnp.concatenate([b_proj, zero5]),
        jnp.concatenate([jnp.sum(W_proj, axis=1), zero5]),
        jnp.concatenate([route_prior_scale[None],
                         residual_scale[None], zero5, jnp.zeros((1,))]),
        jnp.zeros((QPAD,), jnp.float32),
    ])
    caug = jnp.concatenate(
        [-2.0 * block_centers,
         jnp.sum(block_centers * block_centers, axis=1, keepdims=True),
         jnp.zeros((NB, QPAD - 4), jnp.float32)], axis=1)
    down_all = down_w.transpose(1, 0, 2).reshape(h, RANK * NB)
    up_all = up_w.reshape(RANK * NB, h)

    grid = rows // ROW_TILE

    out = pl.pallas_call(
        _fused_kernel,
        grid=(grid,),
        in_specs=[
            pl.BlockSpec((ROW_TILE, h), lambda i: (i, 0)),
            pl.BlockSpec((ROW_TILE, NB), lambda i: (i, 0)),
            pl.BlockSpec((QPAD, h), lambda i: (0, 0)),
            pl.BlockSpec((4, QPAD), lambda i: (0, 0)),
            pl.BlockSpec((NB, QPAD), lambda i: (0, 0)),
            pl.BlockSpec((h, RANK * NB), lambda i: (0, 0)),
            pl.BlockSpec((RANK * NB, h), lambda i: (0, 0)),
        ],
        out_specs=pl.BlockSpec((ROW_TILE, h), lambda i: (i, 0)),
        out_shape=jax.ShapeDtypeStruct((rows, h), jnp.float32),
    )(flat, route_prior, wp, sb, caug, down_all, up_all)

    return out.reshape(b, s, h)
